# baseline (speedup 1.0000x reference)
import functools

import jax
import jax.numpy as jnp
from jax import lax
from jax.experimental import pallas as pl
from jax.experimental.pallas import tpu as pltpu
from jax.experimental.pallas import tpu_sc as plsc

K = 8192
D = 32
TN = 1024
TK = 4096
NKC = K // TK

N = 64 * 1024
G = N // TN

NC, NS = 2, 16
NW = NC * NS
CH = 128
B_PER_W = N // NW
NCH = B_PER_W // CH


def _argmin_body(x_ref, ct_ref, out_ref):
    x = x_ref[...]
    xsq = jnp.sum(x * x, axis=1, keepdims=True)
    xb = x.astype(jnp.bfloat16)

    def half(h):
        ct = ct_ref[:, pl.ds(h * TK, TK)]
        csq = jnp.sum(ct * ct, axis=0, keepdims=True)
        scores = lax.dot_general(xb, ct.astype(jnp.bfloat16),
                                 (((1,), (0,)), ((), ())),
                                 preferred_element_type=jnp.float32)
        d = (xsq - 2.0 * scores) + csq
        cmin = jnp.min(d, axis=1)
        ii = lax.broadcasted_iota(jnp.int32, (TN, TK), 1)
        masked = jnp.where(d <= cmin[:, None], ii, jnp.int32(K))
        cidx = jnp.min(masked, axis=1) + h * TK
        return cmin, cidx

    bvA, biA = half(0)
    bvB, biB = half(1)
    r1 = bvA.astype(jnp.bfloat16).astype(jnp.float32)
    out_ref[0, 0, :] = jnp.where(bvB < r1, biB, biA)


def _argmin_indices(xf, ct):
    idx3 = pl.pallas_call(
        _argmin_body,
        grid=(G,),
        in_specs=[
            pl.BlockSpec((TN, D), lambda g: (g, 0)),
            pl.BlockSpec((D, K), lambda g: (0, 0)),
        ],
        out_specs=pl.BlockSpec((1, 1, TN), lambda g: (g, 0, 0)),
        out_shape=jax.ShapeDtypeStruct((G, 1, TN), jnp.int32),
    )(xf, ct)
    return idx3.reshape(-1)


def _sc_gather(codes, idx_flat):
    idx2 = idx_flat.reshape(N // CH, CH)
    mesh = plsc.VectorSubcoreMesh(core_axis_name="c", subcore_axis_name="s",
                                  num_cores=NC, num_subcores=NS)

    @functools.partial(
        pl.kernel, mesh=mesh,
        compiler_params=pltpu.CompilerParams(use_tc_tiling_on_sc=False),
        out_type=jax.ShapeDtypeStruct((N, D), jnp.float32),
        scratch_types=[
            pltpu.VMEM((NCH, CH), jnp.int32),
            pltpu.VMEM((CH, D), jnp.float32),
            pltpu.SemaphoreType.DMA,
        ],
    )
    def gk(codes_hbm, idx_hbm, out_hbm, idx_v, rows_v, sem):
        wid = lax.axis_index("s") * NC + lax.axis_index("c")
        pltpu.sync_copy(idx_hbm.at[pl.ds(wid * NCH, NCH)], idx_v)

        def body(j, carry):
            pltpu.async_copy(codes_hbm.at[idx_v.at[j]], rows_v, sem).wait()
            pltpu.sync_copy(rows_v, out_hbm.at[pl.ds(wid * B_PER_W + j * CH, CH)])
            return carry

        lax.fori_loop(0, NCH, body, 0)

    return gk(codes, idx2)


def kernel(x, codes):
    xf = x.reshape(-1, D)
    ct = codes.T
    i_flat = _argmin_indices(xf, ct)
    quant = _sc_gather(codes, i_flat)
    return quant.reshape(x.shape), i_flat.reshape(x.shape[:-1])


# speedup vs baseline: 1.0226x; 1.0226x over previous
import functools

import jax
import jax.numpy as jnp
from jax import lax
from jax.experimental import pallas as pl
from jax.experimental.pallas import tpu as pltpu
from jax.experimental.pallas import tpu_sc as plsc

K = 8192
D = 32
TN = 1024
TK = 4096
NKC = K // TK

N = 64 * 1024
G = N // TN

NC, NS = 2, 16
NW = NC * NS
CH = 128
B_PER_W = N // NW
NCH = B_PER_W // CH


def _argmin_body(x_ref, ct_ref, out_ref):
    x = x_ref[...]
    xsq = jnp.sum(x * x, axis=1, keepdims=True)
    xb = x.astype(jnp.bfloat16)

    def half(h):
        ct = ct_ref[:, pl.ds(h * TK, TK)]
        csq = jnp.sum(ct * ct, axis=0, keepdims=True)
        scores = lax.dot_general(xb, ct.astype(jnp.bfloat16),
                                 (((1,), (0,)), ((), ())),
                                 preferred_element_type=jnp.float32)
        d = (xsq - 2.0 * scores) + csq
        cmin = jnp.min(d, axis=1)
        cidx = jnp.argmin(d, axis=1).astype(jnp.int32) + h * TK
        return cmin, cidx

    bvA, biA = half(0)
    bvB, biB = half(1)
    r1 = bvA.astype(jnp.bfloat16).astype(jnp.float32)
    out_ref[0, 0, :] = jnp.where(bvB < r1, biB, biA)


def _argmin_indices(xf, ct):
    idx3 = pl.pallas_call(
        _argmin_body,
        grid=(G,),
        in_specs=[
            pl.BlockSpec((TN, D), lambda g: (g, 0)),
            pl.BlockSpec((D, K), lambda g: (0, 0)),
        ],
        out_specs=pl.BlockSpec((1, 1, TN), lambda g: (g, 0, 0)),
        out_shape=jax.ShapeDtypeStruct((G, 1, TN), jnp.int32),
    )(xf, ct)
    return idx3.reshape(-1)


def _sc_gather(codes, idx_flat):
    idx2 = idx_flat.reshape(N // CH, CH)
    mesh = plsc.VectorSubcoreMesh(core_axis_name="c", subcore_axis_name="s",
                                  num_cores=NC, num_subcores=NS)

    @functools.partial(
        pl.kernel, mesh=mesh,
        compiler_params=pltpu.CompilerParams(use_tc_tiling_on_sc=False),
        out_type=jax.ShapeDtypeStruct((N, D), jnp.float32),
        scratch_types=[
            pltpu.VMEM((NCH, CH), jnp.int32),
            pltpu.VMEM((CH, D), jnp.float32),
            pltpu.SemaphoreType.DMA,
        ],
    )
    def gk(codes_hbm, idx_hbm, out_hbm, idx_v, rows_v, sem):
        wid = lax.axis_index("s") * NC + lax.axis_index("c")
        pltpu.sync_copy(idx_hbm.at[pl.ds(wid * NCH, NCH)], idx_v)

        def body(j, carry):
            pltpu.async_copy(codes_hbm.at[idx_v.at[j]], rows_v, sem).wait()
            pltpu.sync_copy(rows_v, out_hbm.at[pl.ds(wid * B_PER_W + j * CH, CH)])
            return carry

        lax.fori_loop(0, NCH, body, 0)

    return gk(codes, idx2)


def kernel(x, codes):
    xf = x.reshape(-1, D)
    ct = codes.T
    i_flat = _argmin_indices(xf, ct)
    quant = _sc_gather(codes, i_flat)
    return quant.reshape(x.shape), i_flat.reshape(x.shape[:-1])
